# baseline (device time: 28835 ns/iter reference)
import jax
import jax.numpy as jnp
from jax import lax
from jax.experimental import pallas as pl
from jax.experimental.pallas import tpu as pltpu

B = 8
SKV = 512
H = 8
D = 64
HD = H * D
SCALE = D ** -0.5


def kernel(Q, K, V):
    Qc = Q.reshape(B, HD, 1)
    Kr = K.reshape(B, SKV, HD)
    Vr = V.reshape(B, SKV, HD)

    def body(q_ref, k_ref, v_ref, o_ref, comm_o, comm_s, send_sems, recv_sems):
        my_x = lax.axis_index("x")
        my_y = lax.axis_index("y")

        gi = lax.broadcasted_iota(jnp.int32, (HD, H), 0)
        hi = lax.broadcasted_iota(jnp.int32, (HD, H), 1)
        w_mask = (gi // D == hi).astype(jnp.float32) * SCALE

        h2 = lax.broadcasted_iota(jnp.int32, (H, HD), 0)
        j2 = lax.broadcasted_iota(jnp.int32, (H, HD), 1)
        o_mask = (j2 // D == h2).astype(jnp.float32)

        j3 = lax.broadcasted_iota(jnp.int32, (HD, D), 0)
        d3 = lax.broadcasted_iota(jnp.int32, (HD, D), 1)
        e_sum = (j3 % D == d3).astype(jnp.float32)

        for b in range(B):
            qb = q_ref[b]
            kb = k_ref[b]
            vb = v_ref[b]
            w = qb * w_mask
            sb = jnp.dot(kb, w, preferred_element_type=jnp.float32)
            mb = jnp.max(sb, axis=0, keepdims=True)
            pb = jnp.exp(sb - mb)
            lb = jnp.sum(pb, axis=0, keepdims=True)
            of = lax.dot_general(
                pb, vb, (((0,), (0,)), ((), ())),
                preferred_element_type=jnp.float32,
            )
            ob = jnp.dot(of * o_mask, e_sum,
                         preferred_element_type=jnp.float32)
            comm_o[0, b] = ob
            comm_s[0, b, 0:1, :] = mb
            comm_s[0, b, 1:2, :] = lb

        peer = (my_x, 1 - my_y)
        rdma_o = pltpu.make_async_remote_copy(
            src_ref=comm_o.at[0], dst_ref=comm_o.at[1],
            send_sem=send_sems.at[0], recv_sem=recv_sems.at[0],
            device_id=peer, device_id_type=pl.DeviceIdType.MESH,
        )
        rdma_s = pltpu.make_async_remote_copy(
            src_ref=comm_s.at[0], dst_ref=comm_s.at[1],
            send_sem=send_sems.at[1], recv_sem=recv_sems.at[1],
            device_id=peer, device_id_type=pl.DeviceIdType.MESH,
        )
        rdma_o.start()
        rdma_s.start()
        rdma_o.wait()
        rdma_s.wait()

        m_a = comm_s[0, :, 0:1, :]
        l_a = comm_s[0, :, 1:2, :]
        m_b = comm_s[1, :, 0:1, :]
        l_b = comm_s[1, :, 1:2, :]
        m_n = jnp.maximum(m_a, m_b)
        alpha = jnp.exp(m_a - m_n)
        beta = jnp.exp(m_b - m_n)
        l_n = alpha * l_a + beta * l_b
        wa = jnp.transpose(alpha / l_n, (0, 2, 1))
        wb = jnp.transpose(beta / l_n, (0, 2, 1))
        o_ref[:, 0, :, :] = comm_o[0] * wa + comm_o[1] * wb

    return pl.pallas_call(
        body,
        out_shape=jax.ShapeDtypeStruct((B, 1, H, D), jnp.float32),
        in_specs=[pl.BlockSpec(memory_space=pltpu.VMEM)] * 3,
        out_specs=pl.BlockSpec(memory_space=pltpu.VMEM),
        scratch_shapes=[
            pltpu.VMEM((2, B, H, D), jnp.float32),
            pltpu.VMEM((2, B, 2, H), jnp.float32),
            pltpu.SemaphoreType.DMA((2,)),
            pltpu.SemaphoreType.DMA((2,)),
        ],
    )(Qc, Kr, Vr)


# device time: 25206 ns/iter; 1.1440x vs baseline; 1.1440x over previous
import jax
import jax.numpy as jnp
from jax import lax
from jax.experimental import pallas as pl
from jax.experimental.pallas import tpu as pltpu

B = 8
SKV = 512
H = 8
D = 64
HD = H * D
SCALE = D ** -0.5


def kernel(Q, K, V):
    Qr = Q.reshape(B, 1, HD)
    Kr = K.reshape(B, SKV, HD)
    Vr = V.reshape(B, SKV, HD)

    def body(q_ref, k_ref, v_ref, o_ref, comm_o, comm_s, send_sems, recv_sems):
        my_x = lax.axis_index("x")
        my_y = lax.axis_index("y")

        h2 = lax.broadcasted_iota(jnp.int32, (H, HD), 0)
        j2 = lax.broadcasted_iota(jnp.int32, (H, HD), 1)
        hd_mask = (j2 // D == h2).astype(jnp.float32)

        j3 = lax.broadcasted_iota(jnp.int32, (HD, D), 0)
        d3 = lax.broadcasted_iota(jnp.int32, (HD, D), 1)
        e_sum = (j3 % D == d3).astype(jnp.float32)

        for b in range(B):
            qb = q_ref[b]
            kb = k_ref[b]
            vb = v_ref[b]
            wt = qb * hd_mask * SCALE
            sb = lax.dot_general(
                wt, kb, (((1,), (1,)), ((), ())),
                preferred_element_type=jnp.float32,
            )
            mb = jnp.max(sb, axis=1, keepdims=True)
            pb = jnp.exp(sb - mb)
            lb = jnp.sum(pb, axis=1, keepdims=True)
            of = jnp.dot(pb, vb, preferred_element_type=jnp.float32)
            ob = jnp.dot(of * hd_mask, e_sum,
                         preferred_element_type=jnp.float32)
            comm_o[0, b] = ob
            comm_s[0, b, :, 0:1] = mb
            comm_s[0, b, :, 1:2] = lb

        peer = (my_x, 1 - my_y)
        rdma_o = pltpu.make_async_remote_copy(
            src_ref=comm_o.at[0], dst_ref=comm_o.at[1],
            send_sem=send_sems.at[0], recv_sem=recv_sems.at[0],
            device_id=peer, device_id_type=pl.DeviceIdType.MESH,
        )
        rdma_s = pltpu.make_async_remote_copy(
            src_ref=comm_s.at[0], dst_ref=comm_s.at[1],
            send_sem=send_sems.at[1], recv_sem=recv_sems.at[1],
            device_id=peer, device_id_type=pl.DeviceIdType.MESH,
        )
        rdma_o.start()
        rdma_s.start()
        rdma_o.wait()
        rdma_s.wait()

        m_a = comm_s[0, :, :, 0:1]
        l_a = comm_s[0, :, :, 1:2]
        m_b = comm_s[1, :, :, 0:1]
        l_b = comm_s[1, :, :, 1:2]
        m_n = jnp.maximum(m_a, m_b)
        alpha = jnp.exp(m_a - m_n)
        beta = jnp.exp(m_b - m_n)
        l_n = alpha * l_a + beta * l_b
        o_ref[:, 0, :, :] = (comm_o[0] * (alpha / l_n)
                             + comm_o[1] * (beta / l_n))

    return pl.pallas_call(
        body,
        out_shape=jax.ShapeDtypeStruct((B, 1, H, D), jnp.float32),
        in_specs=[pl.BlockSpec(memory_space=pltpu.VMEM)] * 3,
        out_specs=pl.BlockSpec(memory_space=pltpu.VMEM),
        scratch_shapes=[
            pltpu.VMEM((2, B, H, D), jnp.float32),
            pltpu.VMEM((2, B, H, 2), jnp.float32),
            pltpu.SemaphoreType.DMA((2,)),
            pltpu.SemaphoreType.DMA((2,)),
        ],
    )(Qr, Kr, Vr)


# device time: 16709 ns/iter; 1.7257x vs baseline; 1.5085x over previous
import jax
import jax.numpy as jnp
from jax import lax
from jax.experimental import pallas as pl
from jax.experimental.pallas import tpu as pltpu

B = 8
SKV = 512
H = 8
D = 64
HD = H * D
SCALE = D ** -0.5


def kernel(Q, K, V):
    Kt = jnp.transpose(K, (0, 2, 3, 1))
    Vt = jnp.transpose(V, (0, 2, 3, 1))

    def body(q_ref, kt_ref, vt_ref, o_ref, comm_o, comm_s, send_sems, recv_sems):
        my_x = lax.axis_index("x")
        my_y = lax.axis_index("y")
        peer = (my_x, 1 - my_y)

        h2 = lax.broadcasted_iota(jnp.int32, (H, HD), 0)
        j2 = lax.broadcasted_iota(jnp.int32, (H, HD), 1)
        hd_mask = (j2 // D == h2).astype(jnp.float32)

        j3 = lax.broadcasted_iota(jnp.int32, (HD, D), 0)
        d3 = lax.broadcasted_iota(jnp.int32, (HD, D), 1)
        e_sum = (j3 % D == d3).astype(jnp.float32)

        for b in range(B):
            qb = q_ref[b, 0]
            qrow = jnp.tile(qb, (1, H))
            wt = qrow * hd_mask * SCALE
            ktb = kt_ref[b].reshape(HD, SKV)
            vtb = vt_ref[b].reshape(HD, SKV)
            sb = jnp.dot(wt, ktb, preferred_element_type=jnp.float32)
            mb = jnp.max(sb, axis=1, keepdims=True)
            pb = jnp.exp(sb - mb)
            lb = jnp.sum(pb, axis=1, keepdims=True)
            of = lax.dot_general(pb, vtb, (((1,), (1,)), ((), ())),
                                 preferred_element_type=jnp.float32)
            ob = jnp.dot(of * hd_mask, e_sum,
                         preferred_element_type=jnp.float32)
            comm_o[0, b] = ob
            comm_s[0, b, :, 0:1] = mb
            comm_s[0, b, :, 1:2] = lb

        barrier_sem = pltpu.get_barrier_semaphore()
        pl.semaphore_signal(barrier_sem, inc=1, device_id=peer,
                            device_id_type=pl.DeviceIdType.MESH)
        pl.semaphore_wait(barrier_sem, 1)

        rdma_o = pltpu.make_async_remote_copy(
            src_ref=comm_o.at[0], dst_ref=comm_o.at[1],
            send_sem=send_sems.at[0], recv_sem=recv_sems.at[0],
            device_id=peer, device_id_type=pl.DeviceIdType.MESH,
        )
        rdma_s = pltpu.make_async_remote_copy(
            src_ref=comm_s.at[0], dst_ref=comm_s.at[1],
            send_sem=send_sems.at[1], recv_sem=recv_sems.at[1],
            device_id=peer, device_id_type=pl.DeviceIdType.MESH,
        )
        rdma_o.start()
        rdma_s.start()
        rdma_o.wait()
        rdma_s.wait()

        m_a = comm_s[0, :, :, 0:1]
        l_a = comm_s[0, :, :, 1:2]
        m_b = comm_s[1, :, :, 0:1]
        l_b = comm_s[1, :, :, 1:2]
        m_n = jnp.maximum(m_a, m_b)
        alpha = jnp.exp(m_a - m_n)
        beta = jnp.exp(m_b - m_n)
        l_n = alpha * l_a + beta * l_b
        o_ref[:, 0, :, :] = (comm_o[0] * (alpha / l_n)
                             + comm_o[1] * (beta / l_n))

    return pl.pallas_call(
        body,
        out_shape=jax.ShapeDtypeStruct((B, 1, H, D), jnp.float32),
        in_specs=[pl.BlockSpec(memory_space=pltpu.VMEM)] * 3,
        out_specs=pl.BlockSpec(memory_space=pltpu.VMEM),
        scratch_shapes=[
            pltpu.VMEM((2, B, H, D), jnp.float32),
            pltpu.VMEM((2, B, H, 2), jnp.float32),
            pltpu.SemaphoreType.DMA((2,)),
            pltpu.SemaphoreType.DMA((2,)),
        ],
        compiler_params=pltpu.CompilerParams(collective_id=0),
    )(Q, Kt, Vt)


# device time: 16287 ns/iter; 1.7704x vs baseline; 1.0259x over previous
import jax
import jax.numpy as jnp
from jax import lax
from jax.experimental import pallas as pl
from jax.experimental.pallas import tpu as pltpu

B = 8
SKV = 512
H = 8
D = 64
HD = H * D
SCALE = D ** -0.5
NCHUNK = 2
BC = B // NCHUNK


def kernel(Q, K, V):
    Kt = jnp.transpose(K, (0, 2, 3, 1))
    Vt = jnp.transpose(V, (0, 2, 3, 1))

    def body(q_ref, kt_ref, vt_ref, o_ref, comm, send_sems, recv_sems):
        my_x = lax.axis_index("x")
        my_y = lax.axis_index("y")
        peer = (my_x, 1 - my_y)

        barrier_sem = pltpu.get_barrier_semaphore()
        pl.semaphore_signal(barrier_sem, inc=1, device_id=peer,
                            device_id_type=pl.DeviceIdType.MESH)

        h2 = lax.broadcasted_iota(jnp.int32, (H, HD), 0)
        j2 = lax.broadcasted_iota(jnp.int32, (H, HD), 1)
        hd_mask = (j2 // D == h2).astype(jnp.float32)

        j3 = lax.broadcasted_iota(jnp.int32, (HD, D), 0)
        d3 = lax.broadcasted_iota(jnp.int32, (HD, D), 1)
        e_sum = (j3 % D == d3).astype(jnp.float32)

        def rdma_chunk(c):
            lo = c * BC
            return pltpu.make_async_remote_copy(
                src_ref=comm.at[0, lo:lo + BC],
                dst_ref=comm.at[1, lo:lo + BC],
                send_sem=send_sems.at[c], recv_sem=recv_sems.at[c],
                device_id=peer, device_id_type=pl.DeviceIdType.MESH,
            )

        def compute_b(b):
            qb = q_ref[b, 0]
            qrow = jnp.tile(qb, (1, H))
            wt = qrow * hd_mask * SCALE
            ktb = kt_ref[b].reshape(HD, SKV)
            vtb = vt_ref[b].reshape(HD, SKV)
            sb = jnp.dot(wt, ktb, preferred_element_type=jnp.float32)
            mb = jnp.max(sb, axis=1, keepdims=True)
            pb = jnp.exp(sb - mb)
            lb = jnp.sum(pb, axis=1, keepdims=True)
            of = lax.dot_general(pb, vtb, (((1,), (1,)), ((), ())),
                                 preferred_element_type=jnp.float32)
            ob = jnp.dot(of * hd_mask, e_sum,
                         preferred_element_type=jnp.float32)
            comm[0, b, :, 0:D] = ob
            comm[0, b, :, D:D + 1] = mb
            comm[0, b, :, D + 1:D + 2] = lb

        rdmas = []
        for c in range(NCHUNK):
            for b in range(c * BC, (c + 1) * BC):
                compute_b(b)
            if c == 0:
                pl.semaphore_wait(barrier_sem, 1)
            r = rdma_chunk(c)
            r.start()
            rdmas.append(r)
        for r in rdmas:
            r.wait()

        o_a = comm[0, :, :, 0:D]
        m_a = comm[0, :, :, D:D + 1]
        l_a = comm[0, :, :, D + 1:D + 2]
        o_b = comm[1, :, :, 0:D]
        m_b = comm[1, :, :, D:D + 1]
        l_b = comm[1, :, :, D + 1:D + 2]
        m_n = jnp.maximum(m_a, m_b)
        alpha = jnp.exp(m_a - m_n)
        beta = jnp.exp(m_b - m_n)
        l_n = alpha * l_a + beta * l_b
        o_ref[:, 0, :, :] = o_a * (alpha / l_n) + o_b * (beta / l_n)

    return pl.pallas_call(
        body,
        out_shape=jax.ShapeDtypeStruct((B, 1, H, D), jnp.float32),
        in_specs=[pl.BlockSpec(memory_space=pltpu.VMEM)] * 3,
        out_specs=pl.BlockSpec(memory_space=pltpu.VMEM),
        scratch_shapes=[
            pltpu.VMEM((2, B, H, D + 2), jnp.float32),
            pltpu.SemaphoreType.DMA((NCHUNK,)),
            pltpu.SemaphoreType.DMA((NCHUNK,)),
        ],
        compiler_params=pltpu.CompilerParams(collective_id=0),
    )(Q, Kt, Vt)


# device time: 14508 ns/iter; 1.9875x vs baseline; 1.1226x over previous
import jax
import jax.numpy as jnp
from jax import lax
from jax.experimental import pallas as pl
from jax.experimental.pallas import tpu as pltpu

B = 8
SKV = 512
H = 8
D = 64
HD = H * D
SCALE = D ** -0.5
NCHUNK = 2
BC = B // NCHUNK


def _hd_mask():
    h2 = lax.broadcasted_iota(jnp.int32, (H, HD), 0)
    j2 = lax.broadcasted_iota(jnp.int32, (H, HD), 1)
    return (j2 // D == h2).astype(jnp.float32)


def kernel(Q, K, V):
    Kt = jnp.transpose(K, (0, 2, 3, 1))
    Vt = jnp.transpose(V, (0, 2, 3, 1))

    def s_body(q_ref, kt_ref, p_ref, stat_ref):
        hd_mask = _hd_mask()
        for b in range(B):
            qb = q_ref[b, 0]
            qrow = jnp.tile(qb, (1, H))
            wt = qrow * hd_mask * SCALE
            ktb = kt_ref[b].reshape(HD, SKV)
            sb = jnp.dot(wt, ktb, preferred_element_type=jnp.float32)
            mb = jnp.max(sb, axis=1, keepdims=True)
            pb = jnp.exp(sb - mb)
            lb = jnp.sum(pb, axis=1, keepdims=True)
            p_ref[b] = pb
            stat_ref[b, :, 0:1] = mb
            stat_ref[b, :, 1:2] = lb

    P, stats = pl.pallas_call(
        s_body,
        out_shape=(
            jax.ShapeDtypeStruct((B, H, SKV), jnp.float32),
            jax.ShapeDtypeStruct((B, H, 2), jnp.float32),
        ),
        in_specs=[pl.BlockSpec(memory_space=pltpu.VMEM)] * 2,
        out_specs=(pl.BlockSpec(memory_space=pltpu.VMEM),) * 2,
    )(Q, Kt)

    def c_body(p_ref, stat_ref, vt_ref, o_ref, comm, send_sems, recv_sems):
        my_x = lax.axis_index("x")
        my_y = lax.axis_index("y")
        peer = (my_x, 1 - my_y)

        barrier_sem = pltpu.get_barrier_semaphore()
        pl.semaphore_signal(barrier_sem, inc=1, device_id=peer,
                            device_id_type=pl.DeviceIdType.MESH)

        hd_mask = _hd_mask()
        j3 = lax.broadcasted_iota(jnp.int32, (HD, D), 0)
        d3 = lax.broadcasted_iota(jnp.int32, (HD, D), 1)
        e_sum = (j3 % D == d3).astype(jnp.float32)

        def rdma_chunk(c):
            lo = c * BC
            return pltpu.make_async_remote_copy(
                src_ref=comm.at[0, lo:lo + BC],
                dst_ref=comm.at[1, lo:lo + BC],
                send_sem=send_sems.at[c], recv_sem=recv_sems.at[c],
                device_id=peer, device_id_type=pl.DeviceIdType.MESH,
            )

        rdmas = []
        for c in range(NCHUNK):
            for b in range(c * BC, (c + 1) * BC):
                pb = p_ref[b]
                vtb = vt_ref[b].reshape(HD, SKV)
                of = lax.dot_general(pb, vtb, (((1,), (1,)), ((), ())),
                                     preferred_element_type=jnp.float32)
                ob = jnp.dot(of * hd_mask, e_sum,
                             preferred_element_type=jnp.float32)
                comm[0, b, :, 0:D] = ob
                comm[0, b, :, D:D + 2] = stat_ref[b]
            if c == 0:
                pl.semaphore_wait(barrier_sem, 1)
            r = rdma_chunk(c)
            r.start()
            rdmas.append(r)
        for r in rdmas:
            r.wait()

        o_a = comm[0, :, :, 0:D]
        m_a = comm[0, :, :, D:D + 1]
        l_a = comm[0, :, :, D + 1:D + 2]
        o_b = comm[1, :, :, 0:D]
        m_b = comm[1, :, :, D:D + 1]
        l_b = comm[1, :, :, D + 1:D + 2]
        m_n = jnp.maximum(m_a, m_b)
        alpha = jnp.exp(m_a - m_n)
        beta = jnp.exp(m_b - m_n)
        l_n = alpha * l_a + beta * l_b
        o_ref[:, 0, :, :] = o_a * (alpha / l_n) + o_b * (beta / l_n)

    return pl.pallas_call(
        c_body,
        out_shape=jax.ShapeDtypeStruct((B, 1, H, D), jnp.float32),
        in_specs=[pl.BlockSpec(memory_space=pltpu.VMEM)] * 3,
        out_specs=pl.BlockSpec(memory_space=pltpu.VMEM),
        scratch_shapes=[
            pltpu.VMEM((2, B, H, D + 2), jnp.float32),
            pltpu.SemaphoreType.DMA((NCHUNK,)),
            pltpu.SemaphoreType.DMA((NCHUNK,)),
        ],
        compiler_params=pltpu.CompilerParams(collective_id=0),
    )(P, stats, Vt)
